# baseline (speedup 1.0000x reference)
import functools

import jax
import jax.numpy as jnp
from jax import lax
from jax.experimental import pallas as pl
from jax.experimental.pallas import tpu as pltpu
from jax.experimental.pallas import tpu_sc as plsc

N = 100000
E = 1600000
NSEG = 10000
H = 64
CE, RE = 32, 8

NPAD = 102400
PPAD = 10240
EPAD = 1605632
EPT = EPAD // 16
EPT2 = EPAD // 32
C = 1024
GSUB = 128
SUB = C // GSUB
NBINS = 8
BINSZ = NPAD // NBINS
STRIPE = (BINSZ + 16) // 16
DSTRIPE = NPAD // 16
PSTRIPE = PPAD // 16

ROWS_BLK = 2048
NBLK = NPAD // ROWS_BLK

_mesh = plsc.VectorSubcoreMesh(core_axis_name="c", subcore_axis_name="s")
_f32 = jnp.float32


@functools.partial(
    pl.kernel,
    out_type=jax.ShapeDtypeStruct((2 * NPAD,), _f32),
    mesh=_mesh,
    scratch_types=[
        pltpu.VMEM((SUB, GSUB), jnp.int32),
        pltpu.VMEM((GSUB,), _f32),
        pltpu.SemaphoreType.DMA,
        pltpu.VMEM_SHARED((NPAD,), _f32),
    ],
)
def _deg_kernel(dst_hbm, z1d_hbm, ones_hbm, out_hbm, idxm, ones_v, sem, deg_sh):
    c = lax.axis_index("c")
    s = lax.axis_index("s")
    pltpu.sync_copy(z1d_hbm, deg_sh.at[pl.ds(s * DSTRIPE, DSTRIPE)])
    pltpu.sync_copy(ones_hbm, ones_v)
    plsc.subcore_barrier()

    base = c * (EPAD // 2) + s * EPT2

    def outer(i, carry):
        off = base + i * C
        cps = []
        for j in range(SUB):
            cps.append(
                pltpu.async_copy(
                    dst_hbm.at[pl.ds(off + j * GSUB, GSUB)], idxm.at[j], sem))
        for cp in cps:
            cp.wait()
        for j in range(SUB):
            pltpu.sync_copy(ones_v, deg_sh.at[idxm.at[j]], add=True)
        return carry

    lax.fori_loop(0, EPT2 // C, outer, 0)
    plsc.subcore_barrier()
    pltpu.sync_copy(deg_sh.at[pl.ds(s * DSTRIPE, DSTRIPE)],
                    out_hbm.at[pl.ds(c * NPAD + s * DSTRIPE, DSTRIPE)])


@functools.partial(
    pl.kernel,
    out_type=jax.ShapeDtypeStruct((NPAD, H), _f32),
    mesh=_mesh,
    scratch_types=[
        pltpu.VMEM((C,), jnp.int32),
        pltpu.VMEM((C,), jnp.int32),
        pltpu.VMEM((C,), jnp.int32),
        pltpu.VMEM((C,), jnp.int32),
        pltpu.VMEM((SUB, GSUB), jnp.int32),
        pltpu.VMEM((SUB, GSUB, H), _f32),
        pltpu.SemaphoreType.DMA,
        pltpu.SemaphoreType.DMA((SUB,)),
        pltpu.VMEM_SHARED((BINSZ + 16, H), _f32),
    ],
    compiler_params=pltpu.CompilerParams(use_tc_tiling_on_sc=False),
)
def _acc_kernel(src_hbm, dst_hbm, g_hbm, z2d_hbm, out_hbm,
                dvmA, svmA, dvmB, svmB, locm, rows, semI, semG, acc_sh):
    c = lax.axis_index("c")
    s = lax.axis_index("s")
    trash = BINSZ + s
    NI = EPT // C

    def load_idx(off, dvm, svm):
        pltpu.async_copy(dst_hbm.at[pl.ds(off, C)], dvm, semI)
        pltpu.async_copy(src_hbm.at[pl.ds(off, C)], svm, semI)

    def drain_idx(dvm, svm):
        pltpu.make_async_copy(dst_hbm.at[pl.ds(0, C)], dvm, semI).wait()
        pltpu.make_async_copy(src_hbm.at[pl.ds(0, C)], svm, semI).wait()

    for b in range(NBINS // 2):
        binbase = (c * (NBINS // 2) + b) * BINSZ
        pltpu.sync_copy(z2d_hbm, acc_sh.at[pl.ds(s * STRIPE, STRIPE)])
        plsc.subcore_barrier()

        ebase = s * EPT

        def process(i, dvm, svm):
            off = ebase + i * C
            for j in range(SUB):
                def maskit(k, carry2, j=j):
                    d = dvm[pl.ds(j * GSUB + k * 16, 16)]
                    loc = d - binbase
                    inb = (loc >= 0) & (loc < BINSZ)
                    locm[j, pl.ds(k * 16, 16)] = jnp.where(inb, loc, trash)
                    return carry2

                lax.fori_loop(0, GSUB // 16, maskit, 0)
            cps = []
            for j in range(SUB):
                cps.append(
                    pltpu.async_copy(
                        g_hbm.at[svm.at[pl.ds(j * GSUB, GSUB)]],
                        rows.at[j], semG.at[j]))
            for j in range(SUB):
                cps[j].wait()
                pltpu.sync_copy(rows.at[j], acc_sh.at[locm.at[j]], add=True)

        load_idx(ebase, dvmA, svmA)

        def outer(k, carry):
            i0 = 2 * k
            load_idx(ebase + (i0 + 1) * C, dvmB, svmB)
            drain_idx(dvmA, svmA)
            process(i0, dvmA, svmA)

            @pl.when(k + 1 < NI // 2)
            def _():
                load_idx(ebase + (i0 + 2) * C, dvmA, svmA)

            drain_idx(dvmB, svmB)
            process(i0 + 1, dvmB, svmB)
            return carry

        lax.fori_loop(0, NI // 2, outer, 0)
        plsc.subcore_barrier()
        pltpu.sync_copy(
            acc_sh.at[pl.ds(s * (BINSZ // 16), BINSZ // 16)],
            out_hbm.at[pl.ds(binbase + s * (BINSZ // 16), BINSZ // 16)])
        plsc.subcore_barrier()


PWIN = 1024


def _pool_body(batch_ref, h3_ref, sums_ref, cnt_ref):
    i = pl.program_id(0)

    @pl.when(i == 0)
    def _():
        sums_ref[...] = jnp.zeros_like(sums_ref)
        cnt_ref[...] = jnp.zeros_like(cnt_ref)

    base = jnp.minimum(batch_ref[0, 0], PPAD - PWIN)
    rel = batch_ref[...] - base
    oneh = (rel == lax.broadcasted_iota(jnp.int32, (ROWS_BLK, PWIN), 1)
            ).astype(_f32)
    local = lax.dot_general(oneh, h3_ref[...], (((0,), (0,)), ((), ())),
                            preferred_element_type=_f32,
                            precision=lax.Precision.HIGHEST)
    lcnt = lax.dot_general(oneh, jnp.ones((ROWS_BLK, 1), _f32),
                           (((0,), (0,)), ((), ())),
                           preferred_element_type=_f32,
                           precision=lax.Precision.HIGHEST)
    sums_ref[pl.ds(base, PWIN), :] += local
    cnt_ref[pl.ds(base, PWIN), :] += lcnt


def _pool(batch_p, h3):
    return pl.pallas_call(
        _pool_body,
        grid=(NBLK,),
        in_specs=[pl.BlockSpec((ROWS_BLK, 1), lambda i: (i, 0)),
                  pl.BlockSpec((ROWS_BLK, H), lambda i: (i, 0))],
        out_specs=[pl.BlockSpec((PPAD, H), lambda i: (0, 0)),
                   pl.BlockSpec((PPAD, 1), lambda i: (0, 0))],
        out_shape=[jax.ShapeDtypeStruct((PPAD, H), _f32),
                   jax.ShapeDtypeStruct((PPAD, 1), _f32)],
    )(batch_p, h3)


def _embed_body(champ_ref, role_ref, team_ref, cnt0_ref, cnt1_ref,
                ct_ref, rt_ref, w1_ref, g1_ref, dinv_ref):
    t1c = jnp.dot(ct_ref[...], w1_ref[0:CE, :], preferred_element_type=_f32, precision=lax.Precision.HIGHEST)
    t1r = jnp.dot(rt_ref[...], w1_ref[CE:CE + RE, :],
                  preferred_element_type=_f32, precision=lax.Precision.HIGHEST)
    ch = champ_ref[...]
    ro = role_ref[...]
    onehc = (ch == lax.broadcasted_iota(jnp.int32, (ROWS_BLK, 170), 1)
             ).astype(_f32)
    onehr = (ro == lax.broadcasted_iota(jnp.int32, (ROWS_BLK, 10), 1)
             ).astype(_f32)
    h1 = (jnp.dot(onehc, t1c, preferred_element_type=_f32, precision=lax.Precision.HIGHEST)
          + jnp.dot(onehr, t1r, preferred_element_type=_f32, precision=lax.Precision.HIGHEST)
          + team_ref[...].astype(_f32) * w1_ref[CE + RE:CE + RE + 1, :])
    cnt = cnt0_ref[...] + cnt1_ref[...]
    dinv = lax.rsqrt(cnt + 1.0)
    dinv_ref[...] = dinv
    g1_ref[...] = dinv * h1


def _embed(champ, role, team, cnt0, cnt1, champ_table, role_table, W1):
    blk1 = pl.BlockSpec((ROWS_BLK, 1), lambda i: (i, 0))
    return pl.pallas_call(
        _embed_body,
        grid=(NBLK,),
        in_specs=[blk1, blk1, blk1, blk1, blk1,
                  pl.BlockSpec((170, CE), lambda i: (0, 0)),
                  pl.BlockSpec((10, RE), lambda i: (0, 0)),
                  pl.BlockSpec((CE + RE + 1, H), lambda i: (0, 0))],
        out_specs=[pl.BlockSpec((ROWS_BLK, H), lambda i: (i, 0)), blk1],
        out_shape=[jax.ShapeDtypeStruct((NPAD, H), _f32),
                   jax.ShapeDtypeStruct((NPAD, 1), _f32)],
    )(champ, role, team, cnt0, cnt1, champ_table, role_table, W1)


def _stats_body(acc_ref, g1_ref, dinv_ref, b1_ref, out1_ref, stat_ref):
    i = pl.program_id(0)
    out1 = dinv_ref[...] * (acc_ref[...] + g1_ref[...]) + b1_ref[...]
    out1_ref[...] = out1
    rid = i * ROWS_BLK + lax.broadcasted_iota(jnp.int32, (ROWS_BLK, H), 0)
    m = jnp.where(rid < N, out1, 0.0)
    sums = jnp.sum(m, axis=0, keepdims=True)
    sq = jnp.sum(m * m, axis=0, keepdims=True)
    upd = jnp.pad(sums, ((0, 7), (0, 0))) + jnp.pad(sq, ((1, 6), (0, 0)))

    @pl.when(i == 0)
    def _():
        stat_ref[...] = jnp.zeros_like(stat_ref)

    stat_ref[...] += upd


def _stats(acc1, g1, dinv, b1):
    return pl.pallas_call(
        _stats_body,
        grid=(NBLK,),
        in_specs=[pl.BlockSpec((ROWS_BLK, H), lambda i: (i, 0)),
                  pl.BlockSpec((ROWS_BLK, H), lambda i: (i, 0)),
                  pl.BlockSpec((ROWS_BLK, 1), lambda i: (i, 0)),
                  pl.BlockSpec((1, H), lambda i: (0, 0))],
        out_specs=[pl.BlockSpec((ROWS_BLK, H), lambda i: (i, 0)),
                   pl.BlockSpec((8, H), lambda i: (0, 0))],
        out_shape=[jax.ShapeDtypeStruct((NPAD, H), _f32),
                   jax.ShapeDtypeStruct((8, H), _f32)],
    )(acc1, g1, dinv, b1)


def _apply_body(out1_ref, stat_ref, dinv_ref, bg_ref, bb_ref, w2_ref, g2_ref):
    mean = stat_ref[0:1, :] / N
    var = stat_ref[1:2, :] / N - mean * mean
    rstd = lax.rsqrt(var + 1e-5)
    h2 = jnp.maximum(
        (out1_ref[...] - mean) * rstd * bg_ref[...] + bb_ref[...], 0.0)
    g2_ref[...] = dinv_ref[...] * jnp.dot(h2, w2_ref[...],
                                          preferred_element_type=_f32, precision=lax.Precision.HIGHEST)


def _apply(out1, stat, dinv, bn1_g, bn1_b, W2):
    return pl.pallas_call(
        _apply_body,
        grid=(NBLK,),
        in_specs=[pl.BlockSpec((ROWS_BLK, H), lambda i: (i, 0)),
                  pl.BlockSpec((8, H), lambda i: (0, 0)),
                  pl.BlockSpec((ROWS_BLK, 1), lambda i: (i, 0)),
                  pl.BlockSpec((1, H), lambda i: (0, 0)),
                  pl.BlockSpec((1, H), lambda i: (0, 0)),
                  pl.BlockSpec((H, H), lambda i: (0, 0))],
        out_specs=pl.BlockSpec((ROWS_BLK, H), lambda i: (i, 0)),
        out_shape=jax.ShapeDtypeStruct((NPAD, H), _f32),
    )(out1, stat, dinv, bn1_g, bn1_b, W2)


def _post2_body(acc_ref, g2_ref, dinv_ref, b2_ref, h3_ref):
    h3_ref[...] = jnp.maximum(
        dinv_ref[...] * (acc_ref[...] + g2_ref[...]) + b2_ref[...], 0.0)


def _post2(acc2, g2, dinv, b2):
    return pl.pallas_call(
        _post2_body,
        grid=(NBLK,),
        in_specs=[pl.BlockSpec((ROWS_BLK, H), lambda i: (i, 0)),
                  pl.BlockSpec((ROWS_BLK, H), lambda i: (i, 0)),
                  pl.BlockSpec((ROWS_BLK, 1), lambda i: (i, 0)),
                  pl.BlockSpec((1, H), lambda i: (0, 0))],
        out_specs=pl.BlockSpec((ROWS_BLK, H), lambda i: (i, 0)),
        out_shape=jax.ShapeDtypeStruct((NPAD, H), _f32),
    )(acc2, g2, dinv, b2)


def _head_body(s0_ref, c0_ref, fw_ref, fb_ref,
               bg_ref, bb_ref, f2w_ref, f2b_ref, o_ref):
    cnt = jnp.maximum(c0_ref[...], 1.0)
    pooled = s0_ref[...] / cnt
    t = jnp.dot(pooled, fw_ref[...], preferred_element_type=_f32, precision=lax.Precision.HIGHEST) + fb_ref[...]
    rid = lax.broadcasted_iota(jnp.int32, (PPAD, H // 2), 0)
    valid = rid < NSEG
    tm = jnp.where(valid, t, 0.0)
    mean = jnp.sum(tm, axis=0, keepdims=True) / NSEG
    var = jnp.sum(tm * tm, axis=0, keepdims=True) / NSEG - mean * mean
    z = jnp.maximum((t - mean) * lax.rsqrt(var + 1e-5) * bg_ref[...]
                    + bb_ref[...], 0.0)
    logits = jnp.dot(z, f2w_ref[...], preferred_element_type=_f32, precision=lax.Precision.HIGHEST) + f2b_ref[...]
    o_ref[...] = 1.0 / (1.0 + jnp.exp(-logits))


def _head(s0, c0, fc1_W, fc1_b, bn2_g, bn2_b, fc2_W, fc2_b):
    full = lambda shape: pl.BlockSpec(shape, lambda: tuple(0 for _ in shape))
    return pl.pallas_call(
        _head_body,
        in_specs=[full((PPAD, H)), full((PPAD, 1)),
                  full((H, H // 2)), full((1, H // 2)),
                  full((1, H // 2)), full((1, H // 2)),
                  full((H // 2, 1)), full((1, 1))],
        out_specs=full((PPAD, 1)),
        out_shape=jax.ShapeDtypeStruct((PPAD, 1), _f32),
    )(s0, c0, fc1_W, fc1_b, bn2_g, bn2_b, fc2_W, fc2_b)


def kernel(x, edge_index, batch, champ_table, role_table, W1, b1, bn1_g,
           bn1_b, W2, b2, fc1_W, fc1_b, bn2_g, bn2_b, fc2_W, fc2_b):
    pad_n = NPAD - N
    pad_e = EPAD - E

    champ = jnp.pad(x[:, 0], (0, pad_n)).reshape(NPAD, 1)
    role = jnp.pad(x[:, 1], (0, pad_n)).reshape(NPAD, 1)
    team = jnp.pad(x[:, 2], (0, pad_n)).reshape(NPAD, 1)
    src = jnp.pad(edge_index[0], (0, pad_e),
                  constant_values=0)
    src = src.at[E:].set(jnp.arange(pad_e, dtype=src.dtype) % 1024)
    dst = jnp.pad(edge_index[1], (0, pad_e), constant_values=NPAD - 1)
    batch_p = jnp.pad(batch, (0, pad_n), constant_values=PPAD - 1)

    z1d = jnp.zeros((DSTRIPE,), _f32)
    z2d = jnp.zeros((STRIPE, H), _f32)
    ones = jnp.ones((GSUB,), _f32)

    cnt2 = _deg_kernel(dst, z1d, ones)
    cnt0 = cnt2[:NPAD].reshape(NPAD, 1)
    cnt1 = cnt2[NPAD:].reshape(NPAD, 1)

    g1, dinv = _embed(champ, role, team, cnt0, cnt1,
                      champ_table, role_table, W1)
    acc1 = _acc_kernel(src, dst, g1, z2d)
    out1, stat = _stats(acc1, g1, dinv, b1.reshape(1, H))
    g2 = _apply(out1, stat, dinv, bn1_g.reshape(1, H), bn1_b.reshape(1, H), W2)
    acc2 = _acc_kernel(src, dst, g2, z2d)
    h3 = _post2(acc2, g2, dinv, b2.reshape(1, H))

    sums, cnts = _pool(batch_p.reshape(NPAD, 1), h3)

    out = _head(sums, cnts, fc1_W, fc1_b.reshape(1, H // 2),
                bn2_g.reshape(1, H // 2), bn2_b.reshape(1, H // 2),
                fc2_W, fc2_b.reshape(1, 1))
    return out[:NSEG]


# speedup vs baseline: 1.5572x; 1.5572x over previous
import functools

import jax
import jax.numpy as jnp
from jax import lax
from jax.experimental import pallas as pl
from jax.experimental.pallas import tpu as pltpu
from jax.experimental.pallas import tpu_sc as plsc

N = 100000
E = 1600000
NSEG = 10000
H = 64
CE, RE = 32, 8

NPAD = 102400
PPAD = 10240
EPAD = 1605632
EPT = EPAD // 16
EPT2 = EPAD // 32
C = 1024
GSUB = 128
SUB = C // GSUB
BHALF = NPAD // 2
QCOL = 16
NQ = H // QCOL
ACCROWS = BHALF + 128
STRIPE = ACCROWS // 16
WSTRIPE = BHALF // 16
DSTRIPE = NPAD // 16
PSTRIPE = PPAD // 16

ROWS_BLK = 2048
NBLK = NPAD // ROWS_BLK

_mesh = plsc.VectorSubcoreMesh(core_axis_name="c", subcore_axis_name="s")
_f32 = jnp.float32


@functools.partial(
    pl.kernel,
    out_type=jax.ShapeDtypeStruct((2 * NPAD,), _f32),
    mesh=_mesh,
    scratch_types=[
        pltpu.VMEM((SUB, GSUB), jnp.int32),
        pltpu.VMEM((GSUB,), _f32),
        pltpu.SemaphoreType.DMA,
        pltpu.VMEM_SHARED((NPAD,), _f32),
    ],
)
def _deg_kernel(dst_hbm, z1d_hbm, ones_hbm, out_hbm, idxm, ones_v, sem, deg_sh):
    c = lax.axis_index("c")
    s = lax.axis_index("s")
    pltpu.sync_copy(z1d_hbm, deg_sh.at[pl.ds(s * DSTRIPE, DSTRIPE)])
    pltpu.sync_copy(ones_hbm, ones_v)
    plsc.subcore_barrier()

    base = c * (EPAD // 2) + s * EPT2

    def outer(i, carry):
        off = base + i * C
        cps = []
        for j in range(SUB):
            cps.append(
                pltpu.async_copy(
                    dst_hbm.at[pl.ds(off + j * GSUB, GSUB)], idxm.at[j], sem))
        for cp in cps:
            cp.wait()
        for j in range(SUB):
            pltpu.sync_copy(ones_v, deg_sh.at[idxm.at[j]], add=True)
        return carry

    lax.fori_loop(0, EPT2 // C, outer, 0)
    plsc.subcore_barrier()
    pltpu.sync_copy(deg_sh.at[pl.ds(s * DSTRIPE, DSTRIPE)],
                    out_hbm.at[pl.ds(c * NPAD + s * DSTRIPE, DSTRIPE)])


@functools.partial(
    pl.kernel,
    out_type=tuple(jax.ShapeDtypeStruct((NPAD, QCOL), _f32)
                   for _ in range(NQ)),
    mesh=_mesh,
    scratch_types=[
        pltpu.VMEM((C,), jnp.int32),
        pltpu.VMEM((C,), jnp.int32),
        pltpu.VMEM((C,), jnp.int32),
        pltpu.VMEM((C,), jnp.int32),
        pltpu.VMEM((SUB, GSUB), jnp.int32),
        pltpu.VMEM((SUB, GSUB, QCOL), _f32),
        pltpu.SemaphoreType.DMA,
        pltpu.SemaphoreType.DMA((SUB,)),
        pltpu.SemaphoreType.DMA,
        pltpu.VMEM_SHARED((ACCROWS, QCOL), _f32),
    ],
    compiler_params=pltpu.CompilerParams(use_tc_tiling_on_sc=False),
)
def _acc_kernel(src_hbm, dst_hbm, g0_hbm, g1_hbm, g2_hbm, g3_hbm, z2d_hbm,
                o0_hbm, o1_hbm, o2_hbm, o3_hbm,
                dvmA, svmA, dvmB, svmB, locm, rows, semI, semG, semS,
                acc_sh):
    c = lax.axis_index("c")
    s = lax.axis_index("s")
    NI = EPT // C
    nodebase = c * BHALF
    gq = [g0_hbm, g1_hbm, g2_hbm, g3_hbm]
    oq = [o0_hbm, o1_hbm, o2_hbm, o3_hbm]

    def load_idx(off, dvm, svm):
        pltpu.async_copy(dst_hbm.at[pl.ds(off, C)], dvm, semI)
        pltpu.async_copy(src_hbm.at[pl.ds(off, C)], svm, semI)

    def drain_idx(dvm, svm):
        pltpu.make_async_copy(dst_hbm.at[pl.ds(0, C)], dvm, semI).wait()
        pltpu.make_async_copy(src_hbm.at[pl.ds(0, C)], svm, semI).wait()

    for q in range(NQ):
        g_hbm = gq[q]
        pltpu.sync_copy(z2d_hbm, acc_sh.at[pl.ds(s * STRIPE, STRIPE)])
        plsc.subcore_barrier()

        ebase = s * EPT

        def process(i, dvm, svm):
            for j in range(SUB):
                def maskit(k, carry2, j=j):
                    d = dvm[pl.ds(j * GSUB + k * 16, 16)]
                    loc = d - nodebase
                    inb = (loc >= 0) & (loc < BHALF)
                    trash = BHALF + (d & 127)
                    locm[j, pl.ds(k * 16, 16)] = jnp.where(inb, loc, trash)
                    return carry2

                lax.fori_loop(0, GSUB // 16, maskit, 0)
            cps = []
            for j in range(SUB):
                cps.append(
                    pltpu.async_copy(
                        g_hbm.at[svm.at[pl.ds(j * GSUB, GSUB)]],
                        rows.at[j], semG.at[j]))
            scps = []
            for j in range(SUB):
                cps[j].wait()
                scps.append(
                    pltpu.async_copy(rows.at[j], acc_sh.at[locm.at[j]],
                                     semS, add=True))
            for cp in scps:
                cp.wait()

        load_idx(ebase, dvmA, svmA)

        def outer(k, carry):
            i0 = 2 * k
            load_idx(ebase + (i0 + 1) * C, dvmB, svmB)
            drain_idx(dvmA, svmA)
            process(i0, dvmA, svmA)

            @pl.when(k + 1 < NI // 2)
            def _():
                load_idx(ebase + (i0 + 2) * C, dvmA, svmA)

            drain_idx(dvmB, svmB)
            process(i0 + 1, dvmB, svmB)
            return carry

        lax.fori_loop(0, NI // 2, outer, 0)
        plsc.subcore_barrier()
        pltpu.sync_copy(
            acc_sh.at[pl.ds(s * WSTRIPE, WSTRIPE)],
            oq[q].at[pl.ds(nodebase + s * WSTRIPE, WSTRIPE)])
        plsc.subcore_barrier()


PWIN = 1024


def _pool_body(batch_ref, h3_ref, sums_ref, cnt_ref):
    i = pl.program_id(0)

    @pl.when(i == 0)
    def _():
        sums_ref[...] = jnp.zeros_like(sums_ref)
        cnt_ref[...] = jnp.zeros_like(cnt_ref)

    base = jnp.minimum(batch_ref[0, 0], PPAD - PWIN)
    rel = batch_ref[...] - base
    oneh = (rel == lax.broadcasted_iota(jnp.int32, (ROWS_BLK, PWIN), 1)
            ).astype(_f32)
    local = lax.dot_general(oneh, h3_ref[...], (((0,), (0,)), ((), ())),
                            preferred_element_type=_f32,
                            precision=lax.Precision.HIGHEST)
    lcnt = lax.dot_general(oneh, jnp.ones((ROWS_BLK, 1), _f32),
                           (((0,), (0,)), ((), ())),
                           preferred_element_type=_f32,
                           precision=lax.Precision.HIGHEST)
    sums_ref[pl.ds(base, PWIN), :] += local
    cnt_ref[pl.ds(base, PWIN), :] += lcnt


def _pool(batch_p, h3):
    return pl.pallas_call(
        _pool_body,
        grid=(NBLK,),
        in_specs=[pl.BlockSpec((ROWS_BLK, 1), lambda i: (i, 0)),
                  pl.BlockSpec((ROWS_BLK, H), lambda i: (i, 0))],
        out_specs=[pl.BlockSpec((PPAD, H), lambda i: (0, 0)),
                   pl.BlockSpec((PPAD, 1), lambda i: (0, 0))],
        out_shape=[jax.ShapeDtypeStruct((PPAD, H), _f32),
                   jax.ShapeDtypeStruct((PPAD, 1), _f32)],
    )(batch_p, h3)


def _embed_body(champ_ref, role_ref, team_ref, cnt0_ref, cnt1_ref,
                ct_ref, rt_ref, w1_ref, gq0_ref, gq1_ref, gq2_ref, gq3_ref,
                dinv_ref):
    t1c = jnp.dot(ct_ref[...], w1_ref[0:CE, :], preferred_element_type=_f32, precision=lax.Precision.HIGHEST)
    t1r = jnp.dot(rt_ref[...], w1_ref[CE:CE + RE, :],
                  preferred_element_type=_f32, precision=lax.Precision.HIGHEST)
    ch = champ_ref[...]
    ro = role_ref[...]
    onehc = (ch == lax.broadcasted_iota(jnp.int32, (ROWS_BLK, 170), 1)
             ).astype(_f32)
    onehr = (ro == lax.broadcasted_iota(jnp.int32, (ROWS_BLK, 10), 1)
             ).astype(_f32)
    h1 = (jnp.dot(onehc, t1c, preferred_element_type=_f32, precision=lax.Precision.HIGHEST)
          + jnp.dot(onehr, t1r, preferred_element_type=_f32, precision=lax.Precision.HIGHEST)
          + team_ref[...].astype(_f32) * w1_ref[CE + RE:CE + RE + 1, :])
    cnt = cnt0_ref[...] + cnt1_ref[...]
    dinv = lax.rsqrt(cnt + 1.0)
    dinv_ref[...] = dinv
    g1 = dinv * h1
    gq0_ref[...] = g1[:, 0 * QCOL:1 * QCOL]
    gq1_ref[...] = g1[:, 1 * QCOL:2 * QCOL]
    gq2_ref[...] = g1[:, 2 * QCOL:3 * QCOL]
    gq3_ref[...] = g1[:, 3 * QCOL:4 * QCOL]


def _embed(champ, role, team, cnt0, cnt1, champ_table, role_table, W1):
    blk1 = pl.BlockSpec((ROWS_BLK, 1), lambda i: (i, 0))
    return pl.pallas_call(
        _embed_body,
        grid=(NBLK,),
        in_specs=[blk1, blk1, blk1, blk1, blk1,
                  pl.BlockSpec((170, CE), lambda i: (0, 0)),
                  pl.BlockSpec((10, RE), lambda i: (0, 0)),
                  pl.BlockSpec((CE + RE + 1, H), lambda i: (0, 0))],
        out_specs=[pl.BlockSpec((ROWS_BLK, QCOL), lambda i: (i, 0))] * NQ
        + [blk1],
        out_shape=[jax.ShapeDtypeStruct((NPAD, QCOL), _f32)] * NQ
        + [jax.ShapeDtypeStruct((NPAD, 1), _f32)],
    )(champ, role, team, cnt0, cnt1, champ_table, role_table, W1)


def _stats_body(a0_ref, a1_ref, a2_ref, a3_ref,
                q0_ref, q1_ref, q2_ref, q3_ref,
                dinv_ref, b1_ref, out1_ref, stat_ref):
    i = pl.program_id(0)
    acc = jnp.concatenate(
        [a0_ref[...], a1_ref[...], a2_ref[...], a3_ref[...]], axis=1)
    g1 = jnp.concatenate(
        [q0_ref[...], q1_ref[...], q2_ref[...], q3_ref[...]], axis=1)
    out1 = dinv_ref[...] * (acc + g1) + b1_ref[...]
    out1_ref[...] = out1
    rid = i * ROWS_BLK + lax.broadcasted_iota(jnp.int32, (ROWS_BLK, H), 0)
    m = jnp.where(rid < N, out1, 0.0)
    sums = jnp.sum(m, axis=0, keepdims=True)
    sq = jnp.sum(m * m, axis=0, keepdims=True)
    upd = jnp.pad(sums, ((0, 7), (0, 0))) + jnp.pad(sq, ((1, 6), (0, 0)))

    @pl.when(i == 0)
    def _():
        stat_ref[...] = jnp.zeros_like(stat_ref)

    stat_ref[...] += upd


def _stats(acc1, g1, dinv, b1):
    return pl.pallas_call(
        _stats_body,
        grid=(NBLK,),
        in_specs=[pl.BlockSpec((ROWS_BLK, QCOL), lambda i: (i, 0))] * 8
        + [pl.BlockSpec((ROWS_BLK, 1), lambda i: (i, 0)),
           pl.BlockSpec((1, H), lambda i: (0, 0))],
        out_specs=[pl.BlockSpec((ROWS_BLK, H), lambda i: (i, 0)),
                   pl.BlockSpec((8, H), lambda i: (0, 0))],
        out_shape=[jax.ShapeDtypeStruct((NPAD, H), _f32),
                   jax.ShapeDtypeStruct((8, H), _f32)],
    )(*acc1, *g1, dinv, b1)


def _apply_body(out1_ref, stat_ref, dinv_ref, bg_ref, bb_ref, w2_ref,
                gq0_ref, gq1_ref, gq2_ref, gq3_ref):
    mean = stat_ref[0:1, :] / N
    var = stat_ref[1:2, :] / N - mean * mean
    rstd = lax.rsqrt(var + 1e-5)
    h2 = jnp.maximum(
        (out1_ref[...] - mean) * rstd * bg_ref[...] + bb_ref[...], 0.0)
    g2 = dinv_ref[...] * jnp.dot(h2, w2_ref[...],
                                 preferred_element_type=_f32,
                                 precision=lax.Precision.HIGHEST)
    gq0_ref[...] = g2[:, 0 * QCOL:1 * QCOL]
    gq1_ref[...] = g2[:, 1 * QCOL:2 * QCOL]
    gq2_ref[...] = g2[:, 2 * QCOL:3 * QCOL]
    gq3_ref[...] = g2[:, 3 * QCOL:4 * QCOL]


def _apply(out1, stat, dinv, bn1_g, bn1_b, W2):
    return pl.pallas_call(
        _apply_body,
        grid=(NBLK,),
        in_specs=[pl.BlockSpec((ROWS_BLK, H), lambda i: (i, 0)),
                  pl.BlockSpec((8, H), lambda i: (0, 0)),
                  pl.BlockSpec((ROWS_BLK, 1), lambda i: (i, 0)),
                  pl.BlockSpec((1, H), lambda i: (0, 0)),
                  pl.BlockSpec((1, H), lambda i: (0, 0)),
                  pl.BlockSpec((H, H), lambda i: (0, 0))],
        out_specs=[pl.BlockSpec((ROWS_BLK, QCOL), lambda i: (i, 0))] * NQ,
        out_shape=[jax.ShapeDtypeStruct((NPAD, QCOL), _f32)] * NQ,
    )(out1, stat, dinv, bn1_g, bn1_b, W2)


def _post2_body(a0_ref, a1_ref, a2_ref, a3_ref,
                q0_ref, q1_ref, q2_ref, q3_ref, dinv_ref, b2_ref, h3_ref):
    acc = jnp.concatenate(
        [a0_ref[...], a1_ref[...], a2_ref[...], a3_ref[...]], axis=1)
    g2 = jnp.concatenate(
        [q0_ref[...], q1_ref[...], q2_ref[...], q3_ref[...]], axis=1)
    h3_ref[...] = jnp.maximum(
        dinv_ref[...] * (acc + g2) + b2_ref[...], 0.0)


def _post2(acc2, g2, dinv, b2):
    return pl.pallas_call(
        _post2_body,
        grid=(NBLK,),
        in_specs=[pl.BlockSpec((ROWS_BLK, QCOL), lambda i: (i, 0))] * 8
        + [pl.BlockSpec((ROWS_BLK, 1), lambda i: (i, 0)),
           pl.BlockSpec((1, H), lambda i: (0, 0))],
        out_specs=pl.BlockSpec((ROWS_BLK, H), lambda i: (i, 0)),
        out_shape=jax.ShapeDtypeStruct((NPAD, H), _f32),
    )(*acc2, *g2, dinv, b2)


def _head_body(s0_ref, c0_ref, fw_ref, fb_ref,
               bg_ref, bb_ref, f2w_ref, f2b_ref, o_ref):
    cnt = jnp.maximum(c0_ref[...], 1.0)
    pooled = s0_ref[...] / cnt
    t = jnp.dot(pooled, fw_ref[...], preferred_element_type=_f32, precision=lax.Precision.HIGHEST) + fb_ref[...]
    rid = lax.broadcasted_iota(jnp.int32, (PPAD, H // 2), 0)
    valid = rid < NSEG
    tm = jnp.where(valid, t, 0.0)
    mean = jnp.sum(tm, axis=0, keepdims=True) / NSEG
    var = jnp.sum(tm * tm, axis=0, keepdims=True) / NSEG - mean * mean
    z = jnp.maximum((t - mean) * lax.rsqrt(var + 1e-5) * bg_ref[...]
                    + bb_ref[...], 0.0)
    logits = jnp.dot(z, f2w_ref[...], preferred_element_type=_f32, precision=lax.Precision.HIGHEST) + f2b_ref[...]
    o_ref[...] = 1.0 / (1.0 + jnp.exp(-logits))


def _head(s0, c0, fc1_W, fc1_b, bn2_g, bn2_b, fc2_W, fc2_b):
    full = lambda shape: pl.BlockSpec(shape, lambda: tuple(0 for _ in shape))
    return pl.pallas_call(
        _head_body,
        in_specs=[full((PPAD, H)), full((PPAD, 1)),
                  full((H, H // 2)), full((1, H // 2)),
                  full((1, H // 2)), full((1, H // 2)),
                  full((H // 2, 1)), full((1, 1))],
        out_specs=full((PPAD, 1)),
        out_shape=jax.ShapeDtypeStruct((PPAD, 1), _f32),
    )(s0, c0, fc1_W, fc1_b, bn2_g, bn2_b, fc2_W, fc2_b)


def kernel(x, edge_index, batch, champ_table, role_table, W1, b1, bn1_g,
           bn1_b, W2, b2, fc1_W, fc1_b, bn2_g, bn2_b, fc2_W, fc2_b):
    pad_n = NPAD - N
    pad_e = EPAD - E

    champ = jnp.pad(x[:, 0], (0, pad_n)).reshape(NPAD, 1)
    role = jnp.pad(x[:, 1], (0, pad_n)).reshape(NPAD, 1)
    team = jnp.pad(x[:, 2], (0, pad_n)).reshape(NPAD, 1)
    src = jnp.pad(edge_index[0], (0, pad_e),
                  constant_values=0)
    src = src.at[E:].set(jnp.arange(pad_e, dtype=src.dtype) % 1024)
    dst = jnp.pad(edge_index[1], (0, pad_e), constant_values=NPAD - 1)
    batch_p = jnp.pad(batch, (0, pad_n), constant_values=PPAD - 1)

    z1d = jnp.zeros((DSTRIPE,), _f32)
    z2d = jnp.zeros((STRIPE, QCOL), _f32)
    ones = jnp.ones((GSUB,), _f32)

    cnt2 = _deg_kernel(dst, z1d, ones)
    cnt0 = cnt2[:NPAD].reshape(NPAD, 1)
    cnt1 = cnt2[NPAD:].reshape(NPAD, 1)

    *g1, dinv = _embed(champ, role, team, cnt0, cnt1,
                       champ_table, role_table, W1)
    acc1 = _acc_kernel(src, dst, g1[0], g1[1], g1[2], g1[3], z2d)
    out1, stat = _stats(acc1, g1, dinv, b1.reshape(1, H))
    g2 = _apply(out1, stat, dinv, bn1_g.reshape(1, H), bn1_b.reshape(1, H), W2)
    acc2 = _acc_kernel(src, dst, g2[0], g2[1], g2[2], g2[3], z2d)
    h3 = _post2(acc2, g2, dinv, b2.reshape(1, H))

    sums, cnts = _pool(batch_p.reshape(NPAD, 1), h3)

    out = _head(sums, cnts, fc1_W, fc1_b.reshape(1, H // 2),
                bn2_g.reshape(1, H // 2), bn2_b.reshape(1, H // 2),
                fc2_W, fc2_b.reshape(1, 1))
    return out[:NSEG]


# speedup vs baseline: 1.7196x; 1.1043x over previous
import functools

import jax
import jax.numpy as jnp
from jax import lax
from jax.experimental import pallas as pl
from jax.experimental.pallas import tpu as pltpu
from jax.experimental.pallas import tpu_sc as plsc

N = 100000
E = 1600000
NSEG = 10000
H = 64
CE, RE = 32, 8

NPAD = 102400
PPAD = 10240
EPAD = 1605632
EPT = EPAD // 16
EPT2 = EPAD // 32
C = 1024
GSUB = 128
SUB = C // GSUB
BHALF = NPAD // 2
QCOL = 16
NQ = H // QCOL
ACCROWS = BHALF + 128
STRIPE = ACCROWS // 16
WSTRIPE = BHALF // 16
DSTRIPE = NPAD // 16
PSTRIPE = PPAD // 16

ROWS_BLK = 2048
NBLK = NPAD // ROWS_BLK

_mesh = plsc.VectorSubcoreMesh(core_axis_name="c", subcore_axis_name="s")
_f32 = jnp.float32


@functools.partial(
    pl.kernel,
    out_type=jax.ShapeDtypeStruct((2 * NPAD,), _f32),
    mesh=_mesh,
    scratch_types=[
        pltpu.VMEM((SUB, GSUB), jnp.int32),
        pltpu.VMEM((GSUB,), _f32),
        pltpu.SemaphoreType.DMA,
        pltpu.VMEM_SHARED((NPAD,), _f32),
    ],
)
def _deg_kernel(dst_hbm, z1d_hbm, ones_hbm, out_hbm, idxm, ones_v, sem, deg_sh):
    c = lax.axis_index("c")
    s = lax.axis_index("s")
    pltpu.sync_copy(z1d_hbm, deg_sh.at[pl.ds(s * DSTRIPE, DSTRIPE)])
    pltpu.sync_copy(ones_hbm, ones_v)
    plsc.subcore_barrier()

    base = c * (EPAD // 2) + s * EPT2

    def outer(i, carry):
        off = base + i * C
        cps = []
        for j in range(SUB):
            cps.append(
                pltpu.async_copy(
                    dst_hbm.at[pl.ds(off + j * GSUB, GSUB)], idxm.at[j], sem))
        for cp in cps:
            cp.wait()
        for j in range(SUB):
            pltpu.sync_copy(ones_v, deg_sh.at[idxm.at[j]], add=True)
        return carry

    lax.fori_loop(0, EPT2 // C, outer, 0)
    plsc.subcore_barrier()
    pltpu.sync_copy(deg_sh.at[pl.ds(s * DSTRIPE, DSTRIPE)],
                    out_hbm.at[pl.ds(c * NPAD + s * DSTRIPE, DSTRIPE)])


@functools.partial(
    pl.kernel,
    out_type=tuple(jax.ShapeDtypeStruct((NPAD, QCOL), _f32)
                   for _ in range(NQ)),
    mesh=_mesh,
    scratch_types=[
        pltpu.VMEM((C,), jnp.int32),
        pltpu.VMEM((C,), jnp.int32),
        pltpu.VMEM((C,), jnp.int32),
        pltpu.VMEM((C,), jnp.int32),
        pltpu.VMEM((SUB, GSUB), jnp.int32),
        pltpu.VMEM((SUB, GSUB, QCOL), _f32),
        pltpu.SemaphoreType.DMA,
        pltpu.SemaphoreType.DMA((SUB,)),
        pltpu.SemaphoreType.DMA,
        pltpu.VMEM_SHARED((ACCROWS, QCOL), _f32),
    ],
    compiler_params=pltpu.CompilerParams(use_tc_tiling_on_sc=False),
)
def _acc_kernel(src_hbm, dst_hbm, g0_hbm, g1_hbm, g2_hbm, g3_hbm, z2d_hbm,
                o0_hbm, o1_hbm, o2_hbm, o3_hbm,
                dvmA, svmA, dvmB, svmB, locm, rows, semI, semG, semS,
                acc_sh):
    c = lax.axis_index("c")
    s = lax.axis_index("s")
    NI = EPT // C
    nodebase = c * BHALF
    gq = [g0_hbm, g1_hbm, g2_hbm, g3_hbm]
    oq = [o0_hbm, o1_hbm, o2_hbm, o3_hbm]

    def load_idx(off, dvm, svm):
        pltpu.async_copy(dst_hbm.at[pl.ds(off, C)], dvm, semI)
        pltpu.async_copy(src_hbm.at[pl.ds(off, C)], svm, semI)

    def drain_idx(dvm, svm):
        pltpu.make_async_copy(dst_hbm.at[pl.ds(0, C)], dvm, semI).wait()
        pltpu.make_async_copy(src_hbm.at[pl.ds(0, C)], svm, semI).wait()

    for q in range(NQ):
        g_hbm = gq[q]
        pltpu.sync_copy(z2d_hbm, acc_sh.at[pl.ds(s * STRIPE, STRIPE)])
        plsc.subcore_barrier()

        ebase = s * EPT

        def process(i, dvm, svm):
            for j in range(SUB):
                for k in range(GSUB // 16):
                    d = dvm[pl.ds(j * GSUB + k * 16, 16)]
                    loc = d - nodebase
                    inb = (loc >= 0) & (loc < BHALF)
                    trash = BHALF + (d & 127)
                    locm[j, pl.ds(k * 16, 16)] = jnp.where(inb, loc, trash)
            cps = []
            for j in range(SUB):
                cps.append(
                    pltpu.async_copy(
                        g_hbm.at[svm.at[pl.ds(j * GSUB, GSUB)]],
                        rows.at[j], semG.at[j]))
            scps = []
            for j in range(SUB):
                cps[j].wait()
                scps.append(
                    pltpu.async_copy(rows.at[j], acc_sh.at[locm.at[j]],
                                     semS, add=True))
            for cp in scps:
                cp.wait()

        load_idx(ebase, dvmA, svmA)

        def outer(k, carry):
            i0 = 2 * k
            load_idx(ebase + (i0 + 1) * C, dvmB, svmB)
            drain_idx(dvmA, svmA)
            process(i0, dvmA, svmA)

            @pl.when(k + 1 < NI // 2)
            def _():
                load_idx(ebase + (i0 + 2) * C, dvmA, svmA)

            drain_idx(dvmB, svmB)
            process(i0 + 1, dvmB, svmB)
            return carry

        lax.fori_loop(0, NI // 2, outer, 0)
        plsc.subcore_barrier()
        pltpu.sync_copy(
            acc_sh.at[pl.ds(s * WSTRIPE, WSTRIPE)],
            oq[q].at[pl.ds(nodebase + s * WSTRIPE, WSTRIPE)])
        plsc.subcore_barrier()


PWIN = 512


def _pool_body(batch_ref, h3_ref, sums_ref, cnt_ref):
    i = pl.program_id(0)

    @pl.when(i == 0)
    def _():
        sums_ref[...] = jnp.zeros_like(sums_ref)
        cnt_ref[...] = jnp.zeros_like(cnt_ref)

    base = jnp.minimum(batch_ref[0, 0], PPAD - PWIN)
    rel = batch_ref[...] - base
    oneh = (rel == lax.broadcasted_iota(jnp.int32, (ROWS_BLK, PWIN), 1)
            ).astype(_f32)
    local = lax.dot_general(oneh, h3_ref[...], (((0,), (0,)), ((), ())),
                            preferred_element_type=_f32,
                            precision=lax.Precision.HIGHEST)
    lcnt = lax.dot_general(oneh, jnp.ones((ROWS_BLK, 1), _f32),
                           (((0,), (0,)), ((), ())),
                           preferred_element_type=_f32,
                           precision=lax.Precision.HIGHEST)
    sums_ref[pl.ds(base, PWIN), :] += local
    cnt_ref[pl.ds(base, PWIN), :] += lcnt


def _pool(batch_p, h3):
    return pl.pallas_call(
        _pool_body,
        grid=(NBLK,),
        in_specs=[pl.BlockSpec((ROWS_BLK, 1), lambda i: (i, 0)),
                  pl.BlockSpec((ROWS_BLK, H), lambda i: (i, 0))],
        out_specs=[pl.BlockSpec((PPAD, H), lambda i: (0, 0)),
                   pl.BlockSpec((PPAD, 1), lambda i: (0, 0))],
        out_shape=[jax.ShapeDtypeStruct((PPAD, H), _f32),
                   jax.ShapeDtypeStruct((PPAD, 1), _f32)],
    )(batch_p, h3)


def _embed_body(champ_ref, role_ref, team_ref, cnt0_ref, cnt1_ref,
                ct_ref, rt_ref, w1_ref, gq0_ref, gq1_ref, gq2_ref, gq3_ref,
                dinv_ref):
    t1c = jnp.dot(ct_ref[...], w1_ref[0:CE, :], preferred_element_type=_f32, precision=lax.Precision.HIGHEST)
    t1r = jnp.dot(rt_ref[...], w1_ref[CE:CE + RE, :],
                  preferred_element_type=_f32, precision=lax.Precision.HIGHEST)
    ch = champ_ref[...]
    ro = role_ref[...]
    onehc = (ch == lax.broadcasted_iota(jnp.int32, (ROWS_BLK, 170), 1)
             ).astype(_f32)
    onehr = (ro == lax.broadcasted_iota(jnp.int32, (ROWS_BLK, 10), 1)
             ).astype(_f32)
    h1 = (jnp.dot(onehc, t1c, preferred_element_type=_f32,
                  precision=lax.Precision.HIGHEST)
          + jnp.dot(onehr, t1r, preferred_element_type=_f32,
                    precision=lax.Precision.HIGHEST)
          + team_ref[...].astype(_f32) * w1_ref[CE + RE:CE + RE + 1, :])
    cnt = cnt0_ref[...] + cnt1_ref[...]
    dinv = lax.rsqrt(cnt + 1.0)
    dinv_ref[...] = dinv
    g1 = dinv * h1
    gq0_ref[...] = g1[:, 0 * QCOL:1 * QCOL]
    gq1_ref[...] = g1[:, 1 * QCOL:2 * QCOL]
    gq2_ref[...] = g1[:, 2 * QCOL:3 * QCOL]
    gq3_ref[...] = g1[:, 3 * QCOL:4 * QCOL]


def _embed(champ, role, team, cnt0, cnt1, champ_table, role_table, W1):
    blk1 = pl.BlockSpec((ROWS_BLK, 1), lambda i: (i, 0))
    return pl.pallas_call(
        _embed_body,
        grid=(NBLK,),
        in_specs=[blk1, blk1, blk1, blk1, blk1,
                  pl.BlockSpec((170, CE), lambda i: (0, 0)),
                  pl.BlockSpec((10, RE), lambda i: (0, 0)),
                  pl.BlockSpec((CE + RE + 1, H), lambda i: (0, 0))],
        out_specs=[pl.BlockSpec((ROWS_BLK, QCOL), lambda i: (i, 0))] * NQ
        + [blk1],
        out_shape=[jax.ShapeDtypeStruct((NPAD, QCOL), _f32)] * NQ
        + [jax.ShapeDtypeStruct((NPAD, 1), _f32)],
    )(champ, role, team, cnt0, cnt1, champ_table, role_table, W1)


def _stats_body(a0_ref, a1_ref, a2_ref, a3_ref,
                q0_ref, q1_ref, q2_ref, q3_ref,
                dinv_ref, b1_ref, out1_ref, stat_ref):
    i = pl.program_id(0)
    acc = jnp.concatenate(
        [a0_ref[...], a1_ref[...], a2_ref[...], a3_ref[...]], axis=1)
    g1 = jnp.concatenate(
        [q0_ref[...], q1_ref[...], q2_ref[...], q3_ref[...]], axis=1)
    out1 = dinv_ref[...] * (acc + g1) + b1_ref[...]
    out1_ref[...] = out1
    rid = i * ROWS_BLK + lax.broadcasted_iota(jnp.int32, (ROWS_BLK, H), 0)
    m = jnp.where(rid < N, out1, 0.0)
    sums = jnp.sum(m, axis=0, keepdims=True)
    sq = jnp.sum(m * m, axis=0, keepdims=True)
    upd = jnp.pad(sums, ((0, 7), (0, 0))) + jnp.pad(sq, ((1, 6), (0, 0)))

    @pl.when(i == 0)
    def _():
        stat_ref[...] = jnp.zeros_like(stat_ref)

    stat_ref[...] += upd


def _stats(acc1, g1, dinv, b1):
    return pl.pallas_call(
        _stats_body,
        grid=(NBLK,),
        in_specs=[pl.BlockSpec((ROWS_BLK, QCOL), lambda i: (i, 0))] * 8
        + [pl.BlockSpec((ROWS_BLK, 1), lambda i: (i, 0)),
           pl.BlockSpec((1, H), lambda i: (0, 0))],
        out_specs=[pl.BlockSpec((ROWS_BLK, H), lambda i: (i, 0)),
                   pl.BlockSpec((8, H), lambda i: (0, 0))],
        out_shape=[jax.ShapeDtypeStruct((NPAD, H), _f32),
                   jax.ShapeDtypeStruct((8, H), _f32)],
    )(*acc1, *g1, dinv, b1)


def _apply_body(out1_ref, stat_ref, dinv_ref, bg_ref, bb_ref, w2_ref,
                gq0_ref, gq1_ref, gq2_ref, gq3_ref):
    mean = stat_ref[0:1, :] / N
    var = stat_ref[1:2, :] / N - mean * mean
    rstd = lax.rsqrt(var + 1e-5)
    h2 = jnp.maximum(
        (out1_ref[...] - mean) * rstd * bg_ref[...] + bb_ref[...], 0.0)
    g2 = dinv_ref[...] * jnp.dot(h2, w2_ref[...],
                                 preferred_element_type=_f32,
                                 precision=lax.Precision.HIGHEST)
    gq0_ref[...] = g2[:, 0 * QCOL:1 * QCOL]
    gq1_ref[...] = g2[:, 1 * QCOL:2 * QCOL]
    gq2_ref[...] = g2[:, 2 * QCOL:3 * QCOL]
    gq3_ref[...] = g2[:, 3 * QCOL:4 * QCOL]


def _apply(out1, stat, dinv, bn1_g, bn1_b, W2):
    return pl.pallas_call(
        _apply_body,
        grid=(NBLK,),
        in_specs=[pl.BlockSpec((ROWS_BLK, H), lambda i: (i, 0)),
                  pl.BlockSpec((8, H), lambda i: (0, 0)),
                  pl.BlockSpec((ROWS_BLK, 1), lambda i: (i, 0)),
                  pl.BlockSpec((1, H), lambda i: (0, 0)),
                  pl.BlockSpec((1, H), lambda i: (0, 0)),
                  pl.BlockSpec((H, H), lambda i: (0, 0))],
        out_specs=[pl.BlockSpec((ROWS_BLK, QCOL), lambda i: (i, 0))] * NQ,
        out_shape=[jax.ShapeDtypeStruct((NPAD, QCOL), _f32)] * NQ,
    )(out1, stat, dinv, bn1_g, bn1_b, W2)


def _post2_body(a0_ref, a1_ref, a2_ref, a3_ref,
                q0_ref, q1_ref, q2_ref, q3_ref, dinv_ref, b2_ref, h3_ref):
    acc = jnp.concatenate(
        [a0_ref[...], a1_ref[...], a2_ref[...], a3_ref[...]], axis=1)
    g2 = jnp.concatenate(
        [q0_ref[...], q1_ref[...], q2_ref[...], q3_ref[...]], axis=1)
    h3_ref[...] = jnp.maximum(
        dinv_ref[...] * (acc + g2) + b2_ref[...], 0.0)


def _post2(acc2, g2, dinv, b2):
    return pl.pallas_call(
        _post2_body,
        grid=(NBLK,),
        in_specs=[pl.BlockSpec((ROWS_BLK, QCOL), lambda i: (i, 0))] * 8
        + [pl.BlockSpec((ROWS_BLK, 1), lambda i: (i, 0)),
           pl.BlockSpec((1, H), lambda i: (0, 0))],
        out_specs=pl.BlockSpec((ROWS_BLK, H), lambda i: (i, 0)),
        out_shape=jax.ShapeDtypeStruct((NPAD, H), _f32),
    )(*acc2, *g2, dinv, b2)


def _head_body(s0_ref, c0_ref, fw_ref, fb_ref,
               bg_ref, bb_ref, f2w_ref, f2b_ref, o_ref):
    cnt = jnp.maximum(c0_ref[...], 1.0)
    pooled = s0_ref[...] / cnt
    t = jnp.dot(pooled, fw_ref[...], preferred_element_type=_f32, precision=lax.Precision.HIGHEST) + fb_ref[...]
    rid = lax.broadcasted_iota(jnp.int32, (PPAD, H // 2), 0)
    valid = rid < NSEG
    tm = jnp.where(valid, t, 0.0)
    mean = jnp.sum(tm, axis=0, keepdims=True) / NSEG
    var = jnp.sum(tm * tm, axis=0, keepdims=True) / NSEG - mean * mean
    z = jnp.maximum((t - mean) * lax.rsqrt(var + 1e-5) * bg_ref[...]
                    + bb_ref[...], 0.0)
    logits = jnp.dot(z, f2w_ref[...], preferred_element_type=_f32, precision=lax.Precision.HIGHEST) + f2b_ref[...]
    o_ref[...] = 1.0 / (1.0 + jnp.exp(-logits))


def _head(s0, c0, fc1_W, fc1_b, bn2_g, bn2_b, fc2_W, fc2_b):
    full = lambda shape: pl.BlockSpec(shape, lambda: tuple(0 for _ in shape))
    return pl.pallas_call(
        _head_body,
        in_specs=[full((PPAD, H)), full((PPAD, 1)),
                  full((H, H // 2)), full((1, H // 2)),
                  full((1, H // 2)), full((1, H // 2)),
                  full((H // 2, 1)), full((1, 1))],
        out_specs=full((PPAD, 1)),
        out_shape=jax.ShapeDtypeStruct((PPAD, 1), _f32),
    )(s0, c0, fc1_W, fc1_b, bn2_g, bn2_b, fc2_W, fc2_b)


def kernel(x, edge_index, batch, champ_table, role_table, W1, b1, bn1_g,
           bn1_b, W2, b2, fc1_W, fc1_b, bn2_g, bn2_b, fc2_W, fc2_b):
    pad_n = NPAD - N
    pad_e = EPAD - E

    champ = jnp.pad(x[:, 0], (0, pad_n)).reshape(NPAD, 1)
    role = jnp.pad(x[:, 1], (0, pad_n)).reshape(NPAD, 1)
    team = jnp.pad(x[:, 2], (0, pad_n)).reshape(NPAD, 1)
    src = jnp.pad(edge_index[0], (0, pad_e),
                  constant_values=0)
    src = src.at[E:].set(jnp.arange(pad_e, dtype=src.dtype) % 1024)
    dst = jnp.pad(edge_index[1], (0, pad_e), constant_values=NPAD - 1)
    batch_p = jnp.pad(batch, (0, pad_n), constant_values=PPAD - 1)

    z1d = jnp.zeros((DSTRIPE,), _f32)
    z2d = jnp.zeros((STRIPE, QCOL), _f32)
    ones = jnp.ones((GSUB,), _f32)

    cnt2 = _deg_kernel(dst, z1d, ones)
    cnt0 = cnt2[:NPAD].reshape(NPAD, 1)
    cnt1 = cnt2[NPAD:].reshape(NPAD, 1)

    *g1, dinv = _embed(champ, role, team, cnt0, cnt1,
                       champ_table, role_table, W1)
    acc1 = _acc_kernel(src, dst, g1[0], g1[1], g1[2], g1[3], z2d)
    out1, stat = _stats(acc1, g1, dinv, b1.reshape(1, H))
    g2 = _apply(out1, stat, dinv, bn1_g.reshape(1, H), bn1_b.reshape(1, H), W2)
    acc2 = _acc_kernel(src, dst, g2[0], g2[1], g2[2], g2[3], z2d)
    h3 = _post2(acc2, g2, dinv, b2.reshape(1, H))

    sums, cnts = _pool(batch_p.reshape(NPAD, 1), h3)

    out = _head(sums, cnts, fc1_W, fc1_b.reshape(1, H // 2),
                bn2_g.reshape(1, H // 2), bn2_b.reshape(1, H // 2),
                fc2_W, fc2_b.reshape(1, 1))
    return out[:NSEG]
